# baseline (speedup 1.0000x reference)
import functools

import jax
import jax.numpy as jnp
from jax import lax
from jax.experimental import pallas as pl
from jax.experimental.pallas import tpu as pltpu
from jax.experimental.pallas import tpu_sc as plsc

VOCAB = 100000
D = 64
BATCH = 4096
SEQ = 200
B_TOTAL = BATCH * SEQ

NC = 2
NS = 16
NW = NC * NS

CHUNK = 128
PER_W = B_TOTAL // NW
NCHUNK = PER_W // CHUNK
NBUF = 8
GAHEAD = 4


def _make_sc_gather():
    mesh = plsc.VectorSubcoreMesh(
        core_axis_name="c", subcore_axis_name="s", num_cores=NC, num_subcores=NS
    )

    @functools.partial(
        pl.kernel,
        mesh=mesh,
        out_type=jax.ShapeDtypeStruct((B_TOTAL, D), jnp.float32),
        scratch_types=[
            pltpu.VMEM((NCHUNK, CHUNK), jnp.int32),
            pltpu.VMEM((NBUF, CHUNK, D), jnp.float32),
        ]
        + [pltpu.SemaphoreType.DMA] * NBUF
        + [pltpu.SemaphoreType.DMA] * NBUF,
        compiler_params=pltpu.CompilerParams(use_tc_tiling_on_sc=False),
    )
    def k(code_hbm, table_hbm, out_hbm, idx_v, rows_v, *sems):
        gsem = sems[:NBUF]
        wsem = sems[NBUF:]
        wid = lax.axis_index("s") * NC + lax.axis_index("c")
        base = wid * PER_W
        pltpu.sync_copy(code_hbm.at[wid], idx_v)

        def start_gather(j, b):
            pltpu.async_copy(table_hbm.at[idx_v.at[j]], rows_v.at[b], gsem[b])

        def wait_gather(j, b):
            pltpu.make_async_copy(
                table_hbm.at[idx_v.at[j]], rows_v.at[b], gsem[b]
            ).wait()

        def start_write(j, b):
            pltpu.async_copy(
                rows_v.at[b], out_hbm.at[pl.ds(base + j * CHUNK, CHUNK)], wsem[b]
            )

        def wait_write(j, b):
            pltpu.make_async_copy(
                rows_v.at[b], out_hbm.at[pl.ds(base + j * CHUNK, CHUNK)], wsem[b]
            ).wait()

        for t in range(GAHEAD):
            start_gather(t, t)
        for j in range(NBUF - GAHEAD):
            start_gather(j + GAHEAD, j + GAHEAD)
            wait_gather(j, j)
            start_write(j, j)

        J0 = NBUF - GAHEAD
        M = (NCHUNK - GAHEAD - J0) // NBUF

        def block(i, carry):
            for t in range(NBUF):
                j = J0 + i * NBUF + t
                b = (J0 + t) % NBUF
                bg = (J0 + t + GAHEAD) % NBUF
                wait_write(j + GAHEAD - NBUF, bg)
                start_gather(j + GAHEAD, bg)
                wait_gather(j, b)
                start_write(j, b)
            return carry

        lax.fori_loop(0, M, block, 0)

        for j in range(J0 + M * NBUF, NCHUNK):
            b = j % NBUF
            if j + GAHEAD < NCHUNK:
                bg = (j + GAHEAD) % NBUF
                wait_write(j + GAHEAD - NBUF, bg)
                start_gather(j + GAHEAD, bg)
            wait_gather(j, b)
            start_write(j, b)
        for j in range(NCHUNK - NBUF, NCHUNK):
            wait_write(j, j % NBUF)

    return k


_sc_gather = _make_sc_gather()


def kernel(code, embedding):
    code3 = code.reshape(NW, NCHUNK, CHUNK).astype(jnp.int32)
    out = _sc_gather(code3, embedding)
    return out.reshape(BATCH, SEQ, D)


# speedup vs baseline: 1.0012x; 1.0012x over previous
import functools

import jax
import jax.numpy as jnp
from jax import lax
from jax.experimental import pallas as pl
from jax.experimental.pallas import tpu as pltpu
from jax.experimental.pallas import tpu_sc as plsc

VOCAB = 100000
D = 64
BATCH = 4096
SEQ = 200

NC = 2
NS = 16
NW = NC * NS

PER_W = BATCH // NW
NBUF = 6
GAHEAD = 3


def _make_sc_gather():
    mesh = plsc.VectorSubcoreMesh(
        core_axis_name="c", subcore_axis_name="s", num_cores=NC, num_subcores=NS
    )

    @functools.partial(
        pl.kernel,
        mesh=mesh,
        out_type=jax.ShapeDtypeStruct((BATCH, SEQ, D), jnp.float32),
        scratch_types=[
            pltpu.VMEM((PER_W, SEQ), jnp.int32),
            pltpu.VMEM((NBUF, SEQ, D), jnp.float32),
        ]
        + [pltpu.SemaphoreType.DMA] * NBUF
        + [pltpu.SemaphoreType.DMA] * NBUF,
        compiler_params=pltpu.CompilerParams(use_tc_tiling_on_sc=False),
    )
    def k(code_hbm, table_hbm, out_hbm, idx_v, rows_v, *sems):
        gsem = sems[:NBUF]
        wsem = sems[NBUF:]
        wid = lax.axis_index("s") * NC + lax.axis_index("c")
        bb = wid * PER_W
        pltpu.sync_copy(code_hbm.at[wid], idx_v)

        def start_gather(j, b):
            pltpu.async_copy(table_hbm.at[idx_v.at[j]], rows_v.at[b], gsem[b])

        def wait_gather(j, b):
            pltpu.make_async_copy(
                table_hbm.at[idx_v.at[j]], rows_v.at[b], gsem[b]
            ).wait()

        def start_write(j, b):
            pltpu.async_copy(rows_v.at[b], out_hbm.at[bb + j], wsem[b])

        def wait_write(j, b):
            pltpu.make_async_copy(
                rows_v.at[b], out_hbm.at[bb + j], wsem[b]
            ).wait()

        for t in range(GAHEAD):
            start_gather(t, t)
        for j in range(NBUF - GAHEAD):
            start_gather(j + GAHEAD, j + GAHEAD)
            wait_gather(j, j)
            start_write(j, j)

        J0 = NBUF - GAHEAD
        M = (PER_W - GAHEAD - J0) // NBUF

        def block(i, carry):
            for t in range(NBUF):
                j = J0 + i * NBUF + t
                b = (J0 + t) % NBUF
                bg = (J0 + t + GAHEAD) % NBUF
                wait_write(j + GAHEAD - NBUF, bg)
                start_gather(j + GAHEAD, bg)
                wait_gather(j, b)
                start_write(j, b)
            return carry

        lax.fori_loop(0, M, block, 0)

        for j in range(J0 + M * NBUF, PER_W):
            b = j % NBUF
            if j + GAHEAD < PER_W:
                bg = (j + GAHEAD) % NBUF
                wait_write(j + GAHEAD - NBUF, bg)
                start_gather(j + GAHEAD, bg)
            wait_gather(j, b)
            start_write(j, b)
        for j in range(PER_W - NBUF, PER_W):
            wait_write(j, j % NBUF)

    return k


_sc_gather = _make_sc_gather()


def kernel(code, embedding):
    code3 = code.reshape(NW, PER_W, SEQ).astype(jnp.int32)
    return _sc_gather(code3, embedding)
